# baseline (speedup 1.0000x reference)
import jax
import jax.numpy as jnp
from jax import lax
from jax.experimental import pallas as pl
from jax.experimental.pallas import tpu as pltpu
from jax.experimental.pallas import tpu_sc as plsc

_N_ROWS = 65536
_N_COLS = 512
_ONES_ROWS = 512
_BLOCK_ROWS = 2048

_SC_NC = 1
_NS = 16
_SC_ROWS_PER_W = _ONES_ROWS // (_SC_NC * _NS)


def _sc_ones_body(o_hbm, tpl, sem):
    c = lax.axis_index("c")
    s = lax.axis_index("s")
    wid = s * _SC_NC + c

    def row_body(r, carry):
        for cc in range(_N_COLS // 16):
            tpl[r, pl.ds(cc * 16, 16)] = jnp.full((16,), 1.0, jnp.float32)
        return carry

    lax.fori_loop(0, _SC_ROWS_PER_W, row_body, 0)

    cp = pltpu.make_async_copy(
        tpl, o_hbm.at[pl.ds(wid * _SC_ROWS_PER_W, _SC_ROWS_PER_W), :], sem
    )
    cp.start()
    cp.wait()


def _sc_ones(dtype):
    mesh = plsc.VectorSubcoreMesh(
        core_axis_name="c", subcore_axis_name="s", num_cores=_SC_NC
    )
    run = pl.kernel(
        _sc_ones_body,
        out_type=jax.ShapeDtypeStruct((_ONES_ROWS, _N_COLS), dtype),
        mesh=mesh,
        scratch_types=[
            pltpu.VMEM((_SC_ROWS_PER_W, _N_COLS), jnp.float32),
            pltpu.SemaphoreType.DMA,
        ],
    )
    return run()


def _tc_zeros_body(o_ref):
    o_ref[...] = jnp.zeros(o_ref.shape, jnp.float32)


def _tc_zeros(dtype):
    return pl.pallas_call(
        _tc_zeros_body,
        grid=(_N_ROWS // _BLOCK_ROWS,),
        out_specs=pl.BlockSpec((_BLOCK_ROWS, _N_COLS), lambda i: (i, 0)),
        out_shape=jax.ShapeDtypeStruct((_N_ROWS, _N_COLS), dtype),
    )()


def _merge_body(z_ref, ones_ref, o_ref):
    del z_ref
    o_ref[...] = ones_ref[...]


def _merge(zeros, ones_block):
    return pl.pallas_call(
        _merge_body,
        grid=(1,),
        in_specs=[
            pl.BlockSpec(memory_space=pl.ANY),
            pl.BlockSpec((_ONES_ROWS, _N_COLS), lambda i: (0, 0)),
        ],
        out_specs=pl.BlockSpec((_ONES_ROWS, _N_COLS), lambda i: (0, 0)),
        out_shape=jax.ShapeDtypeStruct((_N_ROWS, _N_COLS), zeros.dtype),
        input_output_aliases={0: 0},
    )(zeros, ones_block)


def kernel(x):
    zeros = _tc_zeros(x.dtype)
    ones_block = _sc_ones(x.dtype)
    return _merge(zeros, ones_block)


# speedup vs baseline: 1.0052x; 1.0052x over previous
import jax
import jax.numpy as jnp
from jax import lax
from jax.experimental import pallas as pl
from jax.experimental.pallas import tpu as pltpu
from jax.experimental.pallas import tpu_sc as plsc

_N_ROWS = 65536
_N_COLS = 512
_ONES_ROWS = 512
_BLOCK_ROWS = 2048

_SC_NC = 1
_NS = 16
_SC_ROWS_PER_W = _ONES_ROWS // (_SC_NC * _NS)


def _sc_ones_body(o_hbm, tpl, sem):
    c = lax.axis_index("c")
    s = lax.axis_index("s")
    wid = s * _SC_NC + c

    def row_body(r, carry):
        for cc in range(_N_COLS // 16):
            tpl[r, pl.ds(cc * 16, 16)] = jnp.full((16,), 1.0, jnp.float32)
        return carry

    lax.fori_loop(0, _SC_ROWS_PER_W, row_body, 0)

    cp = pltpu.make_async_copy(
        tpl, o_hbm.at[pl.ds(wid * _SC_ROWS_PER_W, _SC_ROWS_PER_W), :], sem
    )
    cp.start()
    cp.wait()


def _sc_ones(dtype):
    mesh = plsc.VectorSubcoreMesh(
        core_axis_name="c", subcore_axis_name="s", num_cores=_SC_NC
    )
    run = pl.kernel(
        _sc_ones_body,
        out_type=jax.ShapeDtypeStruct((_ONES_ROWS, _N_COLS), dtype),
        mesh=mesh,
        scratch_types=[
            pltpu.VMEM((_SC_ROWS_PER_W, _N_COLS), jnp.float32),
            pltpu.SemaphoreType.DMA,
        ],
    )
    return run()


def _tc_zeros_body(o_ref):
    o_ref[...] = jnp.zeros(o_ref.shape, jnp.float32)


def _tc_zeros(dtype):
    return pl.pallas_call(
        _tc_zeros_body,
        grid=(_N_ROWS // _BLOCK_ROWS,),
        out_specs=pl.BlockSpec((_BLOCK_ROWS, _N_COLS), lambda i: (i, 0)),
        out_shape=jax.ShapeDtypeStruct((_N_ROWS, _N_COLS), dtype),
    )()


def _merge_body(z_ref, ones_ref, o_ref):
    del z_ref
    o_ref[...] = ones_ref[...]


def _merge(zeros, ones_block):
    return pl.pallas_call(
        _merge_body,
        grid=(1,),
        in_specs=[
            pl.BlockSpec(memory_space=pl.ANY),
            pl.BlockSpec((_ONES_ROWS, _N_COLS), lambda i: (0, 0)),
        ],
        out_specs=pl.BlockSpec((_ONES_ROWS, _N_COLS), lambda i: (0, 0)),
        out_shape=jax.ShapeDtypeStruct((_N_ROWS, _N_COLS), zeros.dtype),
        input_output_aliases={0: 0},
    )(zeros, ones_block)


def kernel(x):
    ones_block = _sc_ones(x.dtype)
    zeros = _tc_zeros(x.dtype)
    return _merge(zeros, ones_block)


# speedup vs baseline: 1.0090x; 1.0038x over previous
import jax
import jax.numpy as jnp
from jax import lax
from jax.experimental import pallas as pl
from jax.experimental.pallas import tpu as pltpu
from jax.experimental.pallas import tpu_sc as plsc

_N_ROWS = 65536
_N_COLS = 512
_ONES_ROWS = 512
_BLOCK_ROWS = 2048

_SC_NC = 1
_NS = 16
_SC_ROWS_PER_W = _ONES_ROWS // (_SC_NC * _NS)


def _sc_ones_body(o_hbm, tpl, sem):
    c = lax.axis_index("c")
    s = lax.axis_index("s")
    wid = s * _SC_NC + c

    def row_body(r, carry):
        for cc in range(_N_COLS // 16):
            tpl[r, pl.ds(cc * 16, 16)] = jnp.full((16,), 1.0, jnp.float32)
        return carry

    lax.fori_loop(0, _SC_ROWS_PER_W, row_body, 0)

    cp = pltpu.make_async_copy(
        tpl, o_hbm.at[pl.ds(wid * _SC_ROWS_PER_W, _SC_ROWS_PER_W), :], sem
    )
    cp.start()
    cp.wait()


def _sc_ones(dtype):
    mesh = plsc.VectorSubcoreMesh(
        core_axis_name="c", subcore_axis_name="s", num_cores=_SC_NC
    )
    run = pl.kernel(
        _sc_ones_body,
        out_type=jax.ShapeDtypeStruct((_ONES_ROWS, _N_COLS), dtype),
        mesh=mesh,
        scratch_types=[
            pltpu.VMEM((_SC_ROWS_PER_W, _N_COLS), jnp.float32),
            pltpu.SemaphoreType.DMA,
        ],
    )
    return run()


def _tc_zeros_body(o_ref):
    o_ref[...] = jnp.zeros(o_ref.shape, jnp.float32)


def _tc_zeros(dtype):
    return pl.pallas_call(
        _tc_zeros_body,
        grid=(_N_ROWS // _BLOCK_ROWS - 1,),
        out_specs=pl.BlockSpec((_BLOCK_ROWS, _N_COLS), lambda i: (i + 1, 0)),
        out_shape=jax.ShapeDtypeStruct((_N_ROWS, _N_COLS), dtype),
    )()


def _merge_body(z_ref, ones_ref, o_ref):
    del z_ref
    o_ref[0:_ONES_ROWS, :] = ones_ref[...]
    o_ref[_ONES_ROWS:_BLOCK_ROWS, :] = jnp.zeros(
        (_BLOCK_ROWS - _ONES_ROWS, _N_COLS), jnp.float32
    )


def _merge(zeros, ones_block):
    return pl.pallas_call(
        _merge_body,
        grid=(1,),
        in_specs=[
            pl.BlockSpec(memory_space=pl.ANY),
            pl.BlockSpec((_ONES_ROWS, _N_COLS), lambda i: (0, 0)),
        ],
        out_specs=pl.BlockSpec((_BLOCK_ROWS, _N_COLS), lambda i: (0, 0)),
        out_shape=jax.ShapeDtypeStruct((_N_ROWS, _N_COLS), zeros.dtype),
        input_output_aliases={0: 0},
    )(zeros, ones_block)


def kernel(x):
    ones_block = _sc_ones(x.dtype)
    zeros = _tc_zeros(x.dtype)
    return _merge(zeros, ones_block)
